# initial kernel scaffold (unmeasured)
import jax
import jax.numpy as jnp
from jax import lax
from jax.experimental import pallas as pl
from jax.experimental.pallas import tpu as pltpu

N_DEV = 4


def kernel(t, W):
    m_per, k = t.shape
    _, n = W.shape
    blk = m_per // N_DEV

    def body(t_ref, w_ref, out_ref, comm_ref, send_sems, recv_sems):
        p = lax.axis_index("i")
        left = lax.rem(p + N_DEV - 1, N_DEV)
        right = lax.rem(p + 1, N_DEV)

        barrier_sem = pltpu.get_barrier_semaphore()
        for nbr in (left, right):
            pl.semaphore_signal(
                barrier_sem, inc=1,
                device_id=(nbr,), device_id_type=pl.DeviceIdType.MESH,
            )
        pl.semaphore_wait(barrier_sem, 2)

        def t_block(c):
            return t_ref[pl.ds(c * blk, blk), :]

        def hop(h, src_slot, dst_slot):
            rdma = pltpu.make_async_remote_copy(
                src_ref=comm_ref.at[src_slot],
                dst_ref=comm_ref.at[dst_slot],
                send_sem=send_sems.at[h % 4],
                recv_sem=recv_sems.at[h % 4],
                device_id=(right,),
                device_id_type=pl.DeviceIdType.MESH,
            )
            rdma.start()
            rdma.wait()

        comm_ref[3, :, :] = t_block(p)
        hop(0, 3, 0)
        c1 = lax.rem(p + N_DEV - 1, N_DEV)
        comm_ref[0, :, :] = comm_ref[0, :, :] + t_block(c1)
        hop(1, 0, 1)
        c2 = lax.rem(p + N_DEV - 2, N_DEV)
        comm_ref[1, :, :] = comm_ref[1, :, :] + t_block(c2)
        hop(2, 1, 2)
        q = lax.rem(p + 1, N_DEV)
        acc = comm_ref[2, :, :] + t_block(q)

        blk_out = jnp.dot(acc, w_ref[:, :], preferred_element_type=jnp.float32)
        out_ref[pl.ds(q * blk, blk), :] = blk_out
        comm_ref[2, :, :] = blk_out

        hop(3, 2, 3)
        out_ref[pl.ds(p * blk, blk), :] = comm_ref[3, :, :]
        hop(4, 3, 0)
        c4 = lax.rem(p + N_DEV - 1, N_DEV)
        out_ref[pl.ds(c4 * blk, blk), :] = comm_ref[0, :, :]
        hop(5, 0, 1)
        c5 = lax.rem(p + N_DEV - 2, N_DEV)
        out_ref[pl.ds(c5 * blk, blk), :] = comm_ref[1, :, :]

    return pl.pallas_call(
        body,
        out_shape=jax.ShapeDtypeStruct((m_per, n), jnp.float32),
        in_specs=[
            pl.BlockSpec(memory_space=pltpu.VMEM),
            pl.BlockSpec(memory_space=pltpu.VMEM),
        ],
        out_specs=pl.BlockSpec(memory_space=pltpu.VMEM),
        scratch_shapes=[
            pltpu.VMEM((4, blk, n), jnp.float32),
            pltpu.SemaphoreType.DMA((4,)),
            pltpu.SemaphoreType.DMA((4,)),
        ],
        compiler_params=pltpu.CompilerParams(collective_id=0),
    )(t, W)


# baseline (device time: 312669 ns/iter reference)
import jax
import jax.numpy as jnp
from jax import lax
from jax.experimental import pallas as pl
from jax.experimental.pallas import tpu as pltpu

N_DEV = 4


def kernel(t, W):
    m_per, k = t.shape
    _, n = W.shape
    blk = m_per // N_DEV

    def body(t_ref, w_ref, out_ref, comm_ref, send_sems, recv_sems):
        p = lax.axis_index("i")
        left = lax.rem(p + N_DEV - 1, N_DEV)
        right = lax.rem(p + 1, N_DEV)

        barrier_sem = pltpu.get_barrier_semaphore()
        for nbr in (left, right):
            pl.semaphore_signal(
                barrier_sem, inc=1,
                device_id=(nbr,), device_id_type=pl.DeviceIdType.MESH,
            )
        pl.semaphore_wait(barrier_sem, 2)

        def t_block(c):
            return t_ref[pl.ds(c * blk, blk), :]

        def hop(h, src_slot, dst_slot):
            rdma = pltpu.make_async_remote_copy(
                src_ref=comm_ref.at[src_slot],
                dst_ref=comm_ref.at[dst_slot],
                send_sem=send_sems.at[h % 4],
                recv_sem=recv_sems.at[h % 4],
                device_id=(right,),
                device_id_type=pl.DeviceIdType.MESH,
            )
            rdma.start()
            rdma.wait()

        comm_ref[3, :, :] = t_block(p)
        hop(0, 3, 0)
        c1 = lax.rem(p + N_DEV - 1, N_DEV)
        comm_ref[0, :, :] = comm_ref[0, :, :] + t_block(c1)
        hop(1, 0, 1)
        c2 = lax.rem(p + N_DEV - 2, N_DEV)
        comm_ref[1, :, :] = comm_ref[1, :, :] + t_block(c2)
        hop(2, 1, 2)
        q = lax.rem(p + 1, N_DEV)
        acc = comm_ref[2, :, :] + t_block(q)

        blk_out = jnp.dot(acc, w_ref[:, :], preferred_element_type=jnp.float32)
        out_ref[pl.ds(q * blk, blk), :] = blk_out
        comm_ref[2, :, :] = blk_out

        hop(3, 2, 3)
        out_ref[pl.ds(p * blk, blk), :] = comm_ref[3, :, :]
        hop(4, 3, 0)
        c4 = lax.rem(p + N_DEV - 1, N_DEV)
        out_ref[pl.ds(c4 * blk, blk), :] = comm_ref[0, :, :]
        hop(5, 0, 1)
        c5 = lax.rem(p + N_DEV - 2, N_DEV)
        out_ref[pl.ds(c5 * blk, blk), :] = comm_ref[1, :, :]

    return pl.pallas_call(
        body,
        out_shape=jax.ShapeDtypeStruct((m_per, n), jnp.float32),
        in_specs=[
            pl.BlockSpec(memory_space=pltpu.VMEM),
            pl.BlockSpec(memory_space=pltpu.VMEM),
        ],
        out_specs=pl.BlockSpec(memory_space=pltpu.VMEM),
        scratch_shapes=[
            pltpu.VMEM((4, blk, n), jnp.float32),
            pltpu.SemaphoreType.DMA((4,)),
            pltpu.SemaphoreType.DMA((4,)),
        ],
        compiler_params=pltpu.CompilerParams(
            collective_id=0,
            vmem_limit_bytes=100 * 1024 * 1024,
        ),
    )(t, W)


# device time: 110206 ns/iter; 2.8371x vs baseline; 2.8371x over previous
import jax
import jax.numpy as jnp
from jax import lax
from jax.experimental import pallas as pl
from jax.experimental.pallas import tpu as pltpu

N_DEV = 4


def kernel(t, W):
    m_per, k = t.shape
    _, n = W.shape
    blk = m_per // N_DEV
    half = k // 2

    def body(t_ref, w_ref, out_ref, cw_ref, ccw_ref,
             cw_send, cw_recv, ccw_send, ccw_recv):
        p = lax.axis_index("i")
        left = lax.rem(p + N_DEV - 1, N_DEV)
        right = lax.rem(p + 1, N_DEV)

        barrier_sem = pltpu.get_barrier_semaphore()
        for nbr in (left, right):
            pl.semaphore_signal(
                barrier_sem, inc=1,
                device_id=(nbr,), device_id_type=pl.DeviceIdType.MESH,
            )
        pl.semaphore_wait(barrier_sem, 2)

        def t_cw(c):
            return t_ref[pl.ds(lax.rem(c, N_DEV) * blk, blk), 0:half]

        def t_ccw(c):
            return t_ref[pl.ds(lax.rem(c, N_DEV) * blk, blk), half:k]

        def hop_pair(h, src_slot, dst_slot):
            cw = pltpu.make_async_remote_copy(
                src_ref=cw_ref.at[src_slot],
                dst_ref=cw_ref.at[dst_slot],
                send_sem=cw_send.at[h % 4],
                recv_sem=cw_recv.at[h % 4],
                device_id=(right,),
                device_id_type=pl.DeviceIdType.MESH,
            )
            ccw = pltpu.make_async_remote_copy(
                src_ref=ccw_ref.at[src_slot],
                dst_ref=ccw_ref.at[dst_slot],
                send_sem=ccw_send.at[h % 4],
                recv_sem=ccw_recv.at[h % 4],
                device_id=(left,),
                device_id_type=pl.DeviceIdType.MESH,
            )
            cw.start()
            ccw.start()
            cw.wait()
            ccw.wait()

        bf16 = jnp.bfloat16
        f32 = jnp.float32

        cw_ref[3, :, :] = t_cw(p).astype(bf16)
        ccw_ref[3, :, :] = t_ccw(p + 2).astype(bf16)
        hop_pair(0, 3, 0)
        cw_ref[0, :, :] = (cw_ref[0, :, :].astype(f32) + t_cw(p + 3)).astype(bf16)
        ccw_ref[0, :, :] = (ccw_ref[0, :, :].astype(f32) + t_ccw(p + 3)).astype(bf16)
        hop_pair(1, 0, 1)
        cw_ref[1, :, :] = (cw_ref[1, :, :].astype(f32) + t_cw(p + 2)).astype(bf16)
        ccw_ref[1, :, :] = (ccw_ref[1, :, :].astype(f32) + t_ccw(p)).astype(bf16)
        hop_pair(2, 1, 2)
        q = lax.rem(p + 1, N_DEV)
        s_l = cw_ref[2, :, :].astype(f32) + t_cw(p + 1)
        s_r = ccw_ref[2, :, :].astype(f32) + t_ccw(p + 1)

        s_blk = jnp.concatenate([s_l, s_r], axis=1).astype(bf16)
        blk_out = jnp.dot(s_blk, w_ref[:, :].astype(bf16),
                          preferred_element_type=f32)
        out_ref[pl.ds(q * blk, blk), :] = blk_out

        cw_ref[2, :, :] = blk_out[:, 0:half].astype(bf16)
        ccw_ref[2, :, :] = blk_out[:, half:k].astype(bf16)
        hop_pair(3, 2, 3)
        out_ref[pl.ds(p * blk, blk), 0:half] = cw_ref[3, :, :].astype(f32)
        out_ref[pl.ds(lax.rem(p + 2, N_DEV) * blk, blk), half:k] = (
            ccw_ref[3, :, :].astype(f32))
        hop_pair(4, 3, 0)
        out_ref[pl.ds(lax.rem(p + 3, N_DEV) * blk, blk), 0:half] = (
            cw_ref[0, :, :].astype(f32))
        out_ref[pl.ds(lax.rem(p + 3, N_DEV) * blk, blk), half:k] = (
            ccw_ref[0, :, :].astype(f32))
        hop_pair(5, 0, 1)
        out_ref[pl.ds(lax.rem(p + 2, N_DEV) * blk, blk), 0:half] = (
            cw_ref[1, :, :].astype(f32))
        out_ref[pl.ds(p * blk, blk), half:k] = ccw_ref[1, :, :].astype(f32)

    return pl.pallas_call(
        body,
        out_shape=jax.ShapeDtypeStruct((m_per, n), jnp.float32),
        in_specs=[
            pl.BlockSpec(memory_space=pltpu.VMEM),
            pl.BlockSpec(memory_space=pltpu.VMEM),
        ],
        out_specs=pl.BlockSpec(memory_space=pltpu.VMEM),
        scratch_shapes=[
            pltpu.VMEM((4, blk, half), jnp.bfloat16),
            pltpu.VMEM((4, blk, half), jnp.bfloat16),
            pltpu.SemaphoreType.DMA((4,)),
            pltpu.SemaphoreType.DMA((4,)),
            pltpu.SemaphoreType.DMA((4,)),
            pltpu.SemaphoreType.DMA((4,)),
        ],
        compiler_params=pltpu.CompilerParams(
            collective_id=0,
            vmem_limit_bytes=100 * 1024 * 1024,
        ),
    )(t, W)


# device time: 98721 ns/iter; 3.1672x vs baseline; 1.1163x over previous
import jax
import jax.numpy as jnp
from jax import lax
from jax.experimental import pallas as pl
from jax.experimental.pallas import tpu as pltpu

N_DEV = 4
SUB = 2


def kernel(t, W):
    m_per, k = t.shape
    _, n = W.shape
    blk = m_per // N_DEV
    half = k // 2
    rows = blk // SUB

    SRC = [3, 0, 1, 2, 3, 0]
    DST = [0, 1, 2, 3, 0, 1]
    N_HOPS = 6

    def body(t_ref, w_ref, out_ref, cw_ref, ccw_ref,
             cw_send, cw_recv, ccw_send, ccw_recv):
        p = lax.axis_index("i")
        left = lax.rem(p + N_DEV - 1, N_DEV)
        right = lax.rem(p + 1, N_DEV)

        barrier_sem = pltpu.get_barrier_semaphore()
        for nbr in (left, right):
            pl.semaphore_signal(
                barrier_sem, inc=1,
                device_id=(nbr,), device_id_type=pl.DeviceIdType.MESH,
            )
        pl.semaphore_wait(barrier_sem, 2)

        bf16 = jnp.bfloat16
        f32 = jnp.float32

        def t_cw(c, sub):
            return t_ref[pl.ds(lax.rem(c, N_DEV) * blk + sub * rows, rows),
                         0:half]

        def t_ccw(c, sub):
            return t_ref[pl.ds(lax.rem(c, N_DEV) * blk + sub * rows, rows),
                         half:k]

        dsc = {}
        for h in range(N_HOPS):
            for sub in range(SUB):
                r0 = sub * rows
                i = h * SUB + sub
                dsc["cw", h, sub] = pltpu.make_async_remote_copy(
                    src_ref=cw_ref.at[SRC[h], pl.ds(r0, rows), :],
                    dst_ref=cw_ref.at[DST[h], pl.ds(r0, rows), :],
                    send_sem=cw_send.at[i],
                    recv_sem=cw_recv.at[i],
                    device_id=(right,),
                    device_id_type=pl.DeviceIdType.MESH,
                )
                dsc["ccw", h, sub] = pltpu.make_async_remote_copy(
                    src_ref=ccw_ref.at[SRC[h], pl.ds(r0, rows), :],
                    dst_ref=ccw_ref.at[DST[h], pl.ds(r0, rows), :],
                    send_sem=ccw_send.at[i],
                    recv_sem=ccw_recv.at[i],
                    device_id=(left,),
                    device_id_type=pl.DeviceIdType.MESH,
                )

        def start(h, sub):
            dsc["cw", h, sub].start()
            dsc["ccw", h, sub].start()

        def wait_recv(h, sub):
            dsc["cw", h, sub].wait_recv()
            dsc["ccw", h, sub].wait_recv()

        for sub in range(SUB):
            r0 = sub * rows
            cw_ref[3, pl.ds(r0, rows), :] = t_cw(p, sub).astype(bf16)
            ccw_ref[3, pl.ds(r0, rows), :] = t_ccw(p + 2, sub).astype(bf16)
            start(0, sub)

        cw_acc = [p + 3, p + 2]
        ccw_acc = [p + 3, p]
        for h in (1, 2):
            slot = DST[h - 1]
            for sub in range(SUB):
                r0 = sub * rows
                wait_recv(h - 1, sub)
                cw_ref[slot, pl.ds(r0, rows), :] = (
                    cw_ref[slot, pl.ds(r0, rows), :].astype(f32)
                    + t_cw(cw_acc[h - 1], sub)
                ).astype(bf16)
                ccw_ref[slot, pl.ds(r0, rows), :] = (
                    ccw_ref[slot, pl.ds(r0, rows), :].astype(f32)
                    + t_ccw(ccw_acc[h - 1], sub)
                ).astype(bf16)
                start(h, sub)

        q = lax.rem(p + 1, N_DEV)
        w_bf = w_ref[:, :].astype(bf16)
        for sub in range(SUB):
            r0 = sub * rows
            wait_recv(2, sub)
            s_l = cw_ref[2, pl.ds(r0, rows), :].astype(f32) + t_cw(p + 1, sub)
            s_r = ccw_ref[2, pl.ds(r0, rows), :].astype(f32) + t_ccw(p + 1, sub)
            s_sub = jnp.concatenate([s_l, s_r], axis=1).astype(bf16)
            o = jnp.dot(s_sub, w_bf, preferred_element_type=f32)
            out_ref[pl.ds(q * blk + r0, rows), :] = o
            cw_ref[2, pl.ds(r0, rows), :] = o[:, 0:half].astype(bf16)
            ccw_ref[2, pl.ds(r0, rows), :] = o[:, half:k].astype(bf16)
            start(3, sub)

        cw_store = [p, p + 3, p + 2]
        ccw_store = [p + 2, p + 3, p]
        for h in (4, 5):
            slot = DST[h - 1]
            g = h - 4
            for sub in range(SUB):
                r0 = sub * rows
                wait_recv(h - 1, sub)
                start(h, sub)
                out_ref[pl.ds(lax.rem(cw_store[g], N_DEV) * blk + r0, rows),
                        0:half] = cw_ref[slot, pl.ds(r0, rows), :].astype(f32)
                out_ref[pl.ds(lax.rem(ccw_store[g], N_DEV) * blk + r0, rows),
                        half:k] = ccw_ref[slot, pl.ds(r0, rows), :].astype(f32)

        for sub in range(SUB):
            r0 = sub * rows
            wait_recv(5, sub)
            out_ref[pl.ds(lax.rem(cw_store[2], N_DEV) * blk + r0, rows),
                    0:half] = cw_ref[1, pl.ds(r0, rows), :].astype(f32)
            out_ref[pl.ds(lax.rem(ccw_store[2], N_DEV) * blk + r0, rows),
                    half:k] = ccw_ref[1, pl.ds(r0, rows), :].astype(f32)

        for h in range(N_HOPS):
            for sub in range(SUB):
                dsc["cw", h, sub].wait_send()
                dsc["ccw", h, sub].wait_send()

    n_sems = N_HOPS * SUB
    return pl.pallas_call(
        body,
        out_shape=jax.ShapeDtypeStruct((m_per, n), jnp.float32),
        in_specs=[
            pl.BlockSpec(memory_space=pltpu.VMEM),
            pl.BlockSpec(memory_space=pltpu.VMEM),
        ],
        out_specs=pl.BlockSpec(memory_space=pltpu.VMEM),
        scratch_shapes=[
            pltpu.VMEM((4, blk, half), jnp.bfloat16),
            pltpu.VMEM((4, blk, half), jnp.bfloat16),
            pltpu.SemaphoreType.DMA((n_sems,)),
            pltpu.SemaphoreType.DMA((n_sems,)),
            pltpu.SemaphoreType.DMA((n_sems,)),
            pltpu.SemaphoreType.DMA((n_sems,)),
        ],
        compiler_params=pltpu.CompilerParams(
            collective_id=0,
            vmem_limit_bytes=100 * 1024 * 1024,
        ),
    )(t, W)
